# baseline (speedup 1.0000x reference)
import functools
import jax
import jax.numpy as jnp
from jax import lax
from jax.experimental import pallas as pl
from jax.experimental.pallas import tpu as pltpu
from jax.experimental.pallas import tpu_sc as plsc

_NEG = -3.402823466e38


def _h_body(x_ref, w1_ref, b1_ref, w2_ref, b2_ref, o_ref):
    h = jnp.maximum(
        jnp.dot(x_ref[...], w1_ref[...], preferred_element_type=jnp.float32)
        + b1_ref[...], 0.0)
    o_ref[...] = (
        jnp.dot(h, w2_ref[...], preferred_element_type=jnp.float32) + b2_ref[...])


def _embed(x, p):
    N = x.shape[0]
    BLK = 6256
    grid = N // BLK
    full = lambda s: pl.BlockSpec(s, lambda i: tuple(0 for _ in s))
    return pl.pallas_call(
        _h_body,
        grid=(grid,),
        in_specs=[
            pl.BlockSpec((BLK, x.shape[1]), lambda i: (i, 0)),
            full(p['W1'].shape), full((1, 16)), full(p['W2'].shape), full((1, 16)),
        ],
        out_specs=pl.BlockSpec((BLK, 16), lambda i: (i, 0)),
        out_shape=jax.ShapeDtypeStruct((N, 16), jnp.float32),
    )(x, p['W1'], p['b1'].reshape(1, 16), p['W2'], p['b2'].reshape(1, 16))


def _e_body(a_ref, w1_ref, b1_ref, w2_ref, b2_ref, w3_ref, b3_ref, o_ref):
    e = jnp.maximum(
        jnp.dot(a_ref[...], w1_ref[...], preferred_element_type=jnp.float32)
        + b1_ref[...], 0.0)
    e = jnp.maximum(
        jnp.dot(e, w2_ref[...], preferred_element_type=jnp.float32)
        + b2_ref[...], 0.0)
    o_ref[...] = (
        jnp.dot(e, w3_ref[...], preferred_element_type=jnp.float32) + b3_ref[...])


def _edge_mlp(edge_attr, p):
    E = edge_attr.shape[0]
    R = E // 16
    BLK = 4000
    grid = R // BLK
    ea = edge_attr.reshape(R, 128)
    eye = jnp.eye(16, dtype=jnp.float32)
    w1 = jnp.kron(eye, p['W1'])
    w2 = jnp.kron(eye, p['W2'])
    w3 = jnp.kron(eye, p['W3'])
    b1 = jnp.tile(p['b1'], 16).reshape(1, 256)
    b2 = jnp.tile(p['b2'], 16).reshape(1, 256)
    b3 = jnp.tile(p['b3'], 16).reshape(1, 256)
    full = lambda s: pl.BlockSpec(s, lambda i: tuple(0 for _ in s))
    out = pl.pallas_call(
        _e_body,
        grid=(grid,),
        in_specs=[
            pl.BlockSpec((BLK, 128), lambda i: (i, 0)),
            full(w1.shape), full((1, 256)),
            full(w2.shape), full((1, 256)),
            full(w3.shape), full((1, 256)),
        ],
        out_specs=pl.BlockSpec((BLK, 256), lambda i: (i, 0)),
        out_shape=jax.ShapeDtypeStruct((R, 256), jnp.float32),
    )(ea, w1, b1, w2, b2, w3, b3)
    return out.reshape(E, 16)


def _z_body(h_ref, agg_ref, scale_ref, w1_ref, b1_ref, w2_ref, b2_ref, o_ref):
    agg = agg_ref[...]
    agg = jnp.where(agg > _NEG * 0.5, jnp.where(agg >= 0, agg, 0.01 * agg), 0.0)
    z = scale_ref[0, 0] * h_ref[...] + agg
    z = jnp.dot(z, w1_ref[...], preferred_element_type=jnp.float32) + b1_ref[...]
    z = jnp.where(z >= 0, z, 0.01 * z)
    o_ref[...] = (
        jnp.dot(z, w2_ref[...], preferred_element_type=jnp.float32) + b2_ref[...])


def _node_update(h, agg_raw, scale, p):
    N = h.shape[0]
    BLK = 6256
    grid = N // BLK
    full = lambda s: pl.BlockSpec(s, lambda i: tuple(0 for _ in s))
    return pl.pallas_call(
        _z_body,
        grid=(grid,),
        in_specs=[
            pl.BlockSpec((BLK, 16), lambda i: (i, 0)),
            pl.BlockSpec((BLK, 16), lambda i: (i, 0)),
            pl.BlockSpec(memory_space=pltpu.SMEM),
            full(p['W1'].shape), full((1, 16)), full(p['W2'].shape), full((1, 16)),
        ],
        out_specs=pl.BlockSpec((BLK, 16), lambda i: (i, 0)),
        out_shape=jax.ShapeDtypeStruct((N, 16), jnp.float32),
    )(h, agg_raw, scale, p['W1'], p['b1'].reshape(1, 16), p['W2'],
      p['b2'].reshape(1, 16))


def _out_body(hp_ref, w1_ref, b1_ref, g_ref, be_ref, w2_ref, b2_ref, y_ref, p_ref):
    hp = jnp.max(hp_ref[...], axis=0)
    hp = jnp.where(hp > _NEG * 0.5, hp, 0.0)
    y = jnp.dot(hp, w1_ref[...], preferred_element_type=jnp.float32) + b1_ref[...]
    y = y * (g_ref[...] / jnp.sqrt(1.0 + 1e-5)) + be_ref[...]
    y = jnp.where(y >= 0, y, 0.01 * y)
    y = jnp.dot(y, w2_ref[...], preferred_element_type=jnp.float32) + b2_ref[...]
    y_ref[...] = y
    p_ref[...] = jax.nn.sigmoid(y)


def _out_mlp(partials, p):
    P, G, _ = partials.shape
    full = lambda s: pl.BlockSpec(s, lambda: tuple(0 for _ in s))
    return pl.pallas_call(
        _out_body,
        in_specs=[full((P, G, 16)), full(p['W1'].shape), full((1, 16)),
                  full((1, 16)), full((1, 16)), full(p['W2'].shape), full((1, 1))],
        out_specs=(full((G, 1)), full((G, 1))),
        out_shape=(jax.ShapeDtypeStruct((G, 1), jnp.float32),
                   jax.ShapeDtypeStruct((G, 1), jnp.float32)),
    )(partials, p['W1'], p['b1'].reshape(1, 16), p['gamma'].reshape(1, 16),
      p['beta'].reshape(1, 16), p['W2'], p['b2'].reshape(1, 1))


_NC, _NS, _NW = 2, 16, 32


def _edge_scatter_max(h, e, ei, n):
    E = ei.shape[1]
    RN = n // _NW
    CE = 4000
    FB = 1024
    CAP = FB + CE + 16
    mesh = plsc.VectorSubcoreMesh(core_axis_name="c", subcore_axis_name="s")

    @functools.partial(
        pl.kernel, mesh=mesh,
        compiler_params=pltpu.CompilerParams(needs_layout_passes=False, use_tc_tiling_on_sc=False),
        out_type=jax.ShapeDtypeStruct((n, 16), jnp.float32),
        scratch_types=[
            pltpu.VMEM((2, CE), jnp.int32),
            pltpu.VMEM((2, CE), jnp.int32),
            pltpu.VMEM((CAP + 16,), jnp.int32),
            pltpu.VMEM((CAP + 16,), jnp.int32),
            pltpu.VMEM((CAP + 16,), jnp.int32),
            pltpu.VMEM((FB, 16), jnp.float32),
            pltpu.VMEM((FB, 16), jnp.float32),
            pltpu.VMEM((RN + 1, 16), jnp.float32),
            pltpu.SemaphoreType.DMA,
            pltpu.SemaphoreType.DMA,
            pltpu.SemaphoreType.DMA,
            pltpu.SemaphoreType.DMA,
            pltpu.SemaphoreType.DMA,
            pltpu.SemaphoreType.DMA,
        ],
    )
    def k(h_hbm, e_hbm, ei_hbm, agg_hbm,
          dst_v, src_v, eid_b, srb_b, dlo_b, erows, hrows, aggl,
          sem1, sem2, sd0, sd1, ss0, ss1):
        wid = lax.axis_index("s") * _NC + lax.axis_index("c")
        lo = wid * RN
        sd = (sd0, sd1)
        ss = (ss0, ss1)

        def start_load(c, b):
            pltpu.async_copy(ei_hbm.at[1, pl.ds(c * CE, CE)], dst_v.at[b], sd[b])
            pltpu.async_copy(ei_hbm.at[0, pl.ds(c * CE, CE)], src_v.at[b], ss[b])

        def wait_load(c, b):
            pltpu.make_async_copy(
                ei_hbm.at[1, pl.ds(c * CE, CE)], dst_v.at[b], sd[b]).wait()
            pltpu.make_async_copy(
                ei_hbm.at[0, pl.ds(c * CE, CE)], src_v.at[b], ss[b]).wait()

        def init_row(i, _):
            aggl[i] = jnp.full((16,), _NEG, jnp.float32)
            return 0
        lax.fori_loop(0, RN + 1, init_row, 0)

        def do_flush():
            c1 = pltpu.async_copy(e_hbm.at[eid_b.at[pl.ds(0, FB)]], erows, sem1)
            c2 = pltpu.async_copy(h_hbm.at[srb_b.at[pl.ds(0, FB)]], hrows, sem2)
            c1.wait()
            c2.wait()

            def upd(j16, _):
                base = j16 * 16
                dvec = dlo_b[pl.ds(base, 16)]
                for u in range(16):
                    d = dvec[u]
                    aggl[d] = jnp.maximum(
                        aggl[d], hrows[base + u] + erows[base + u])
                return 0
            lax.fori_loop(0, FB // 16, upd, 0)

        def flush_step(p):
            do_flush()
            nrem = p - FB

            def mv(k2, _):
                s_ = pl.ds(FB + k2 * 16, 16)
                d_ = pl.ds(k2 * 16, 16)
                eid_b[d_] = eid_b[s_]
                srb_b[d_] = srb_b[s_]
                dlo_b[d_] = dlo_b[s_]
                return 0
            lax.fori_loop(0, (nrem + 15) // 16, mv, 0)
            return nrem

        NCH = E // CE

        def process(c, b, ptr):
            wait_load(c, b)

            @pl.when(c + 1 < NCH)
            def _():
                start_load(c + 1, 1 - b)

            def scan(i, p):
                for u in range(10):
                    off = i * 160 + u * 16
                    dv = dst_v[b, pl.ds(off, 16)]
                    sv = src_v[b, pl.ds(off, 16)]
                    locv = dv - lo
                    m = (locv >= 0) & (locv < RN)
                    cnt = plsc.all_reduce_population_count(m)[0]
                    plsc.store_compressed(dlo_b.at[pl.ds(p, 16)], locv, mask=m)
                    plsc.store_compressed(srb_b.at[pl.ds(p, 16)], sv, mask=m)
                    eidv = c * CE + off + lax.iota(jnp.int32, 16)
                    plsc.store_compressed(eid_b.at[pl.ds(p, 16)], eidv, mask=m)
                    p = p + cnt
                return p
            ptr = lax.fori_loop(0, CE // 160, scan, ptr)
            return lax.while_loop(lambda p: p >= FB, flush_step, ptr)

        def chunk_pair(c2, ptr):
            ptr = process(2 * c2, 0, ptr)
            ptr = process(2 * c2 + 1, 1, ptr)
            return ptr

        start_load(0, 0)
        ptr = lax.fori_loop(0, NCH // 2, chunk_pair, jnp.int32(0))

        def pad(k2, _):
            sl = pl.ds(k2 * 16, 16)
            iv = k2 * 16 + lax.iota(jnp.int32, 16)
            pr = iv >= ptr
            dlo_b[sl] = jnp.where(pr, RN, dlo_b[sl])
            srb_b[sl] = jnp.where(pr, 0, srb_b[sl])
            eid_b[sl] = jnp.where(pr, 0, eid_b[sl])
            return 0
        lax.fori_loop(0, FB // 16, pad, 0)
        do_flush()

        pltpu.sync_copy(aggl.at[pl.ds(0, RN)], agg_hbm.at[pl.ds(lo, RN)])

    return k(h, e, ei)


def _pool_partials(z, batch, g):
    n = z.shape[0]
    RT = n // _NW
    mesh = plsc.VectorSubcoreMesh(core_axis_name="c", subcore_axis_name="s")

    @functools.partial(
        pl.kernel, mesh=mesh,
        compiler_params=pltpu.CompilerParams(needs_layout_passes=False, use_tc_tiling_on_sc=False),
        out_type=jax.ShapeDtypeStruct((_NW, g, 16), jnp.float32),
        scratch_types=[
            pltpu.VMEM((RT, 16), jnp.float32),
            pltpu.VMEM((RT + 16,), jnp.int32),
            pltpu.VMEM((g + 1, 16), jnp.float32),
        ],
    )
    def k(z_hbm, b_hbm, out_hbm, zrows, bat_v, pooll):
        wid = lax.axis_index("s") * _NC + lax.axis_index("c")
        start = wid * RT

        def init_row(i, _):
            pooll[i] = jnp.full((16,), _NEG, jnp.float32)
            return 0
        lax.fori_loop(0, g + 1, init_row, 0)

        pltpu.sync_copy(z_hbm.at[pl.ds(start, RT)], zrows)
        pltpu.sync_copy(b_hbm.at[pl.ds(start, RT)], bat_v.at[pl.ds(0, RT)])

        def upd(j, _):
            b = bat_v[pl.ds(j, 16)][0]
            pooll[b] = jnp.maximum(pooll[b], zrows[j])
            return 0
        lax.fori_loop(0, RT, upd, 0)

        pltpu.sync_copy(pooll.at[pl.ds(0, g)], out_hbm.at[wid])

    return k(z, batch)


def kernel(x, edge_index, batch, edge_attr, params):
    N = x.shape[0]
    G = 1000
    NP = ((N + 8 * _NW - 1) // (8 * _NW)) * (8 * _NW)
    ei = edge_index.astype(jnp.int32)
    batch = batch.astype(jnp.int32)
    x = jnp.pad(x, ((0, NP - N), (0, 0)))
    batch = jnp.pad(batch, (0, NP - N), constant_values=G)

    h = _embed(x, params['embd'])
    e = _edge_mlp(edge_attr, params['edge_mlp'])
    agg_raw = _edge_scatter_max(h, e, ei, NP)
    scale = (1.0 + params['gine']['eps']).reshape(1, 1)
    z = _node_update(h, agg_raw, scale, params['gine'])
    partials = _pool_partials(z, batch, G)
    y, prob = _out_mlp(partials, params['out'])
    return (y, prob)


# speedup vs baseline: 1.0501x; 1.0501x over previous
import functools
import jax
import jax.numpy as jnp
from jax import lax
from jax.experimental import pallas as pl
from jax.experimental.pallas import tpu as pltpu
from jax.experimental.pallas import tpu_sc as plsc

_NEG = -3.402823466e38


def _h_body(x_ref, w1_ref, b1_ref, w2_ref, b2_ref, o_ref):
    h = jnp.maximum(
        jnp.dot(x_ref[...], w1_ref[...], preferred_element_type=jnp.float32)
        + b1_ref[...], 0.0)
    o_ref[...] = (
        jnp.dot(h, w2_ref[...], preferred_element_type=jnp.float32) + b2_ref[...])


def _embed(x, p):
    N = x.shape[0]
    BLK = 6256
    grid = N // BLK
    full = lambda s: pl.BlockSpec(s, lambda i: tuple(0 for _ in s))
    return pl.pallas_call(
        _h_body,
        grid=(grid,),
        in_specs=[
            pl.BlockSpec((BLK, x.shape[1]), lambda i: (i, 0)),
            full(p['W1'].shape), full((1, 16)), full(p['W2'].shape), full((1, 16)),
        ],
        out_specs=pl.BlockSpec((BLK, 16), lambda i: (i, 0)),
        out_shape=jax.ShapeDtypeStruct((N, 16), jnp.float32),
    )(x, p['W1'], p['b1'].reshape(1, 16), p['W2'], p['b2'].reshape(1, 16))


def _e_body(a_ref, w1_ref, b1_ref, w2_ref, b2_ref, w3_ref, b3_ref, o_ref):
    e = jnp.maximum(
        jnp.dot(a_ref[...], w1_ref[...], preferred_element_type=jnp.float32)
        + b1_ref[...], 0.0)
    e = jnp.maximum(
        jnp.dot(e, w2_ref[...], preferred_element_type=jnp.float32)
        + b2_ref[...], 0.0)
    o_ref[...] = (
        jnp.dot(e, w3_ref[...], preferred_element_type=jnp.float32) + b3_ref[...])


def _edge_mlp(edge_attr, p):
    E = edge_attr.shape[0]
    R = E // 16
    BLK = 4000
    grid = R // BLK
    ea = edge_attr.reshape(R, 128)
    eye = jnp.eye(16, dtype=jnp.float32)
    w1 = jnp.kron(eye, p['W1'])
    w2 = jnp.kron(eye, p['W2'])
    w3 = jnp.kron(eye, p['W3'])
    b1 = jnp.tile(p['b1'], 16).reshape(1, 256)
    b2 = jnp.tile(p['b2'], 16).reshape(1, 256)
    b3 = jnp.tile(p['b3'], 16).reshape(1, 256)
    full = lambda s: pl.BlockSpec(s, lambda i: tuple(0 for _ in s))
    out = pl.pallas_call(
        _e_body,
        grid=(grid,),
        in_specs=[
            pl.BlockSpec((BLK, 128), lambda i: (i, 0)),
            full(w1.shape), full((1, 256)),
            full(w2.shape), full((1, 256)),
            full(w3.shape), full((1, 256)),
        ],
        out_specs=pl.BlockSpec((BLK, 256), lambda i: (i, 0)),
        out_shape=jax.ShapeDtypeStruct((R, 256), jnp.float32),
    )(ea, w1, b1, w2, b2, w3, b3)
    return out.reshape(E, 16)


def _z_body(h_ref, agg_ref, scale_ref, w1_ref, b1_ref, w2_ref, b2_ref, o_ref):
    agg = agg_ref[...]
    agg = jnp.where(agg > _NEG * 0.5, jnp.where(agg >= 0, agg, 0.01 * agg), 0.0)
    z = scale_ref[0, 0] * h_ref[...] + agg
    z = jnp.dot(z, w1_ref[...], preferred_element_type=jnp.float32) + b1_ref[...]
    z = jnp.where(z >= 0, z, 0.01 * z)
    o_ref[...] = (
        jnp.dot(z, w2_ref[...], preferred_element_type=jnp.float32) + b2_ref[...])


def _node_update(h, agg_raw, scale, p):
    N = h.shape[0]
    BLK = 6256
    grid = N // BLK
    full = lambda s: pl.BlockSpec(s, lambda i: tuple(0 for _ in s))
    return pl.pallas_call(
        _z_body,
        grid=(grid,),
        in_specs=[
            pl.BlockSpec((BLK, 16), lambda i: (i, 0)),
            pl.BlockSpec((BLK, 16), lambda i: (i, 0)),
            pl.BlockSpec(memory_space=pltpu.SMEM),
            full(p['W1'].shape), full((1, 16)), full(p['W2'].shape), full((1, 16)),
        ],
        out_specs=pl.BlockSpec((BLK, 16), lambda i: (i, 0)),
        out_shape=jax.ShapeDtypeStruct((N, 16), jnp.float32),
    )(h, agg_raw, scale, p['W1'], p['b1'].reshape(1, 16), p['W2'],
      p['b2'].reshape(1, 16))


def _out_body(hp_ref, w1_ref, b1_ref, g_ref, be_ref, w2_ref, b2_ref, y_ref, p_ref):
    hp = jnp.max(hp_ref[...], axis=0)
    hp = jnp.where(hp > _NEG * 0.5, hp, 0.0)
    y = jnp.dot(hp, w1_ref[...], preferred_element_type=jnp.float32) + b1_ref[...]
    y = y * (g_ref[...] / jnp.sqrt(1.0 + 1e-5)) + be_ref[...]
    y = jnp.where(y >= 0, y, 0.01 * y)
    y = jnp.dot(y, w2_ref[...], preferred_element_type=jnp.float32) + b2_ref[...]
    y_ref[...] = y
    p_ref[...] = jax.nn.sigmoid(y)


def _out_mlp(partials, p):
    P, G, _ = partials.shape
    full = lambda s: pl.BlockSpec(s, lambda: tuple(0 for _ in s))
    return pl.pallas_call(
        _out_body,
        in_specs=[full((P, G, 16)), full(p['W1'].shape), full((1, 16)),
                  full((1, 16)), full((1, 16)), full(p['W2'].shape), full((1, 1))],
        out_specs=(full((G, 1)), full((G, 1))),
        out_shape=(jax.ShapeDtypeStruct((G, 1), jnp.float32),
                   jax.ShapeDtypeStruct((G, 1), jnp.float32)),
    )(partials, p['W1'], p['b1'].reshape(1, 16), p['gamma'].reshape(1, 16),
      p['beta'].reshape(1, 16), p['W2'], p['b2'].reshape(1, 1))


_NC, _NS, _NW = 2, 16, 32


def _edge_scatter_max(h, e, ei, n):
    E = ei.shape[1]
    RN = n // _NW
    CE = 4000
    FB = 1024
    CAP = FB + CE + 16
    mesh = plsc.VectorSubcoreMesh(core_axis_name="c", subcore_axis_name="s")

    @functools.partial(
        pl.kernel, mesh=mesh,
        compiler_params=pltpu.CompilerParams(needs_layout_passes=False, use_tc_tiling_on_sc=False),
        out_type=jax.ShapeDtypeStruct((n, 16), jnp.float32),
        scratch_types=[
            pltpu.VMEM((2, CE), jnp.int32),
            pltpu.VMEM((2, CE), jnp.int32),
            pltpu.VMEM((CAP + 16,), jnp.int32),
            pltpu.VMEM((CAP + 16,), jnp.int32),
            pltpu.VMEM((CAP + 16,), jnp.int32),
            pltpu.VMEM((FB, 16), jnp.float32),
            pltpu.VMEM((FB, 16), jnp.float32),
            pltpu.VMEM((RN + 1, 16), jnp.float32),
            pltpu.SemaphoreType.DMA,
            pltpu.SemaphoreType.DMA,
            pltpu.SemaphoreType.DMA,
            pltpu.SemaphoreType.DMA,
            pltpu.SemaphoreType.DMA,
            pltpu.SemaphoreType.DMA,
        ],
    )
    def k(h_hbm, e_hbm, ei_hbm, agg_hbm,
          dst_v, src_v, eid_b, srb_b, dlo_b, erows, hrows, aggl,
          sem1, sem2, sd0, sd1, ss0, ss1):
        wid = lax.axis_index("s") * _NC + lax.axis_index("c")
        lo = wid * RN
        sd = (sd0, sd1)
        ss = (ss0, ss1)

        def start_load(c, b):
            pltpu.async_copy(ei_hbm.at[1, pl.ds(c * CE, CE)], dst_v.at[b], sd[b])
            pltpu.async_copy(ei_hbm.at[0, pl.ds(c * CE, CE)], src_v.at[b], ss[b])

        def wait_load(c, b):
            pltpu.make_async_copy(
                ei_hbm.at[1, pl.ds(c * CE, CE)], dst_v.at[b], sd[b]).wait()
            pltpu.make_async_copy(
                ei_hbm.at[0, pl.ds(c * CE, CE)], src_v.at[b], ss[b]).wait()

        def init_row(i, _):
            aggl[i] = jnp.full((16,), _NEG, jnp.float32)
            return 0
        lax.fori_loop(0, RN + 1, init_row, 0)

        def do_flush():
            c1 = pltpu.async_copy(e_hbm.at[eid_b.at[pl.ds(0, FB)]], erows, sem1)
            c2 = pltpu.async_copy(h_hbm.at[srb_b.at[pl.ds(0, FB)]], hrows, sem2)
            c1.wait()
            c2.wait()

            def upd(j16, _):
                base = j16 * 16
                dvec = dlo_b[pl.ds(base, 16)]
                for u in range(16):
                    d = dvec[u]
                    aggl[d] = jnp.maximum(
                        aggl[d], hrows[base + u] + erows[base + u])
                return 0
            lax.fori_loop(0, FB // 16, upd, 0)

        def flush_step(p):
            do_flush()
            nrem = p - FB

            def mv(k2, _):
                s_ = pl.ds(FB + k2 * 16, 16)
                d_ = pl.ds(k2 * 16, 16)
                eid_b[d_] = eid_b[s_]
                srb_b[d_] = srb_b[s_]
                dlo_b[d_] = dlo_b[s_]
                return 0
            lax.fori_loop(0, (nrem + 15) // 16, mv, 0)
            return nrem

        NCH = E // CE

        def process(c, b, ptr):
            wait_load(c, b)

            @pl.when(c + 1 < NCH)
            def _():
                start_load(c + 1, 1 - b)

            def scan(i, p):
                for u in range(5):
                    off = i * 80 + u * 16
                    dv = dst_v[b, pl.ds(off, 16)]
                    sv = src_v[b, pl.ds(off, 16)]
                    locv = dv - lo
                    m = (locv >= 0) & (locv < RN)
                    cnt = plsc.all_reduce_population_count(m)[0]
                    plsc.store_compressed(dlo_b.at[pl.ds(p, 16)], locv, mask=m)
                    plsc.store_compressed(srb_b.at[pl.ds(p, 16)], sv, mask=m)
                    eidv = c * CE + off + lax.iota(jnp.int32, 16)
                    plsc.store_compressed(eid_b.at[pl.ds(p, 16)], eidv, mask=m)
                    p = p + cnt
                return p
            ptr = lax.fori_loop(0, CE // 80, scan, ptr)
            return lax.while_loop(lambda p: p >= FB, flush_step, ptr)

        def chunk_pair(c2, ptr):
            ptr = process(2 * c2, 0, ptr)
            ptr = process(2 * c2 + 1, 1, ptr)
            return ptr

        start_load(0, 0)
        ptr = lax.fori_loop(0, NCH // 2, chunk_pair, jnp.int32(0))

        def pad(k2, _):
            sl = pl.ds(k2 * 16, 16)
            iv = k2 * 16 + lax.iota(jnp.int32, 16)
            pr = iv >= ptr
            dlo_b[sl] = jnp.where(pr, RN, dlo_b[sl])
            srb_b[sl] = jnp.where(pr, 0, srb_b[sl])
            eid_b[sl] = jnp.where(pr, 0, eid_b[sl])
            return 0
        lax.fori_loop(0, FB // 16, pad, 0)
        do_flush()

        pltpu.sync_copy(aggl.at[pl.ds(0, RN)], agg_hbm.at[pl.ds(lo, RN)])

    return k(h, e, ei)


def _pool_partials(z, batch, g):
    n = z.shape[0]
    RT = n // _NW
    mesh = plsc.VectorSubcoreMesh(core_axis_name="c", subcore_axis_name="s")

    @functools.partial(
        pl.kernel, mesh=mesh,
        compiler_params=pltpu.CompilerParams(needs_layout_passes=False, use_tc_tiling_on_sc=False),
        out_type=jax.ShapeDtypeStruct((_NW, g, 16), jnp.float32),
        scratch_types=[
            pltpu.VMEM((RT, 16), jnp.float32),
            pltpu.VMEM((RT + 16,), jnp.int32),
            pltpu.VMEM((g + 1, 16), jnp.float32),
        ],
    )
    def k(z_hbm, b_hbm, out_hbm, zrows, bat_v, pooll):
        wid = lax.axis_index("s") * _NC + lax.axis_index("c")
        start = wid * RT

        def init_row(i, _):
            pooll[i] = jnp.full((16,), _NEG, jnp.float32)
            return 0
        lax.fori_loop(0, g + 1, init_row, 0)

        pltpu.sync_copy(z_hbm.at[pl.ds(start, RT)], zrows)
        pltpu.sync_copy(b_hbm.at[pl.ds(start, RT)], bat_v.at[pl.ds(0, RT)])

        def upd(j, _):
            b = bat_v[pl.ds(j, 16)][0]
            pooll[b] = jnp.maximum(pooll[b], zrows[j])
            return 0
        lax.fori_loop(0, RT, upd, 0)

        pltpu.sync_copy(pooll.at[pl.ds(0, g)], out_hbm.at[wid])

    return k(z, batch)


def kernel(x, edge_index, batch, edge_attr, params):
    N = x.shape[0]
    G = 1000
    NP = ((N + 8 * _NW - 1) // (8 * _NW)) * (8 * _NW)
    ei = edge_index.astype(jnp.int32)
    batch = batch.astype(jnp.int32)
    x = jnp.pad(x, ((0, NP - N), (0, 0)))
    batch = jnp.pad(batch, (0, NP - N), constant_values=G)

    h = _embed(x, params['embd'])
    e = _edge_mlp(edge_attr, params['edge_mlp'])
    agg_raw = _edge_scatter_max(h, e, ei, NP)
    scale = (1.0 + params['gine']['eps']).reshape(1, 1)
    z = _node_update(h, agg_raw, scale, params['gine'])
    partials = _pool_partials(z, batch, G)
    y, prob = _out_mlp(partials, params['out'])
    return (y, prob)
